# baseline (speedup 1.0000x reference)
import jax
import jax.numpy as jnp
from jax import lax
from jax.experimental import pallas as pl
from jax.experimental.pallas import tpu as pltpu
from jax.experimental.pallas import tpu_sc as plsc

B = 4
S = 8192
C = 768
K = 8
L = 16
NW = 32
CB = C // (NW // B)
GPT = CB // L
CH = 512
NCH = S // CH

_SORT8 = ((0, 1), (2, 3), (4, 5), (6, 7),
          (0, 2), (1, 3), (4, 6), (5, 7),
          (1, 2), (5, 6),
          (0, 4), (1, 5), (2, 6), (3, 7),
          (2, 4), (3, 5),
          (1, 2), (3, 4), (5, 6))

_BITONIC = ((4, (0, 1, 2, 3)), (2, (0, 1, 4, 5)), (1, (0, 2, 4, 6)))


def _sort8(vs):
    vs = list(vs)
    for a, b in _SORT8:
        lo = jnp.minimum(vs[a], vs[b])
        hi = jnp.maximum(vs[a], vs[b])
        vs[a], vs[b] = lo, hi
    return vs


def _bitonic(c, desc):
    c = list(c)
    for d, idxs in _BITONIC:
        for i in idxs:
            lo = jnp.minimum(c[i], c[i + d])
            hi = jnp.maximum(c[i], c[i + d])
            c[i], c[i + d] = (hi, lo) if desc else (lo, hi)
    return c


def _consume_chunk(buf, g, carry):
    def batch_body(bi, st):
        base = bi * 8
        s = _sort8([buf[base + i, pl.ds(g * L, L)] for i in range(8)])
        cmax = [jnp.maximum(st[i], s[i]) for i in range(K)]
        maxs = _bitonic(cmax, desc=True)
        cmin = [jnp.minimum(st[K + i], s[7 - i]) for i in range(K)]
        mins = _bitonic(cmin, desc=False)
        return tuple(maxs) + tuple(mins)

    return lax.fori_loop(0, CH // 8, batch_body, carry, unroll=2)


def _kmink_body(x_hbm, out_hbm, buf0, buf1, state, obuf, sem0, sem1):
    wid = lax.axis_index("s") * 2 + lax.axis_index("c")
    b = wid // (NW // B)
    cb = wid % (NW // B)
    lane = jnp.arange(L, dtype=jnp.int32)
    bufs = (buf0, buf1)
    sems = (sem0, sem1)

    def src(ci):
        return x_hbm.at[b, pl.ds(ci * CH, CH), pl.ds(cb * CB, CB)]

    neg_inf = jnp.full((L,), -jnp.inf, jnp.float32)
    pos_inf = jnp.full((L,), jnp.inf, jnp.float32)

    def init_state(g, _):
        for i in range(K):
            state[g, i] = neg_inf
            state[g, K + i] = pos_inf
        return 0

    lax.fori_loop(0, GPT, init_state, 0)

    pltpu.async_copy(src(0), bufs[0], sems[0]).wait()

    def do_chunk_pair(ci, _):
        for bb in range(2):
            ce = ci * 2 + bb
            nxt = 1 - bb
            have_next = ce + 1 < NCH

            @pl.when(have_next)
            def _():
                pltpu.async_copy(src(ce + 1), bufs[nxt], sems[nxt])

            def do_group(g, _):
                carry = tuple(state[g, i] for i in range(2 * K))
                carry = _consume_chunk(bufs[bb], g, carry)
                for i in range(2 * K):
                    state[g, i] = carry[i]
                return 0

            lax.fori_loop(0, GPT, do_group, 0)

            @pl.when(have_next)
            def _():
                pltpu.make_async_copy(src(ce + 1), bufs[nxt], sems[nxt]).wait()
        return 0

    lax.fori_loop(0, NCH // 2, do_chunk_pair, 0)

    def write_group(g, _):
        for i in range(K):
            plsc.store_scatter(obuf, [lane, jnp.full((L,), i, jnp.int32)],
                               state[g, K + i])
            plsc.store_scatter(obuf, [lane, jnp.full((L,), K + i, jnp.int32)],
                               state[g, i])
        pltpu.sync_copy(obuf, out_hbm.at[b, pl.ds(cb * CB + g * L, L), :])
        return 0

    lax.fori_loop(0, GPT, write_group, 0)


@jax.jit
def kernel(input):
    mesh = plsc.VectorSubcoreMesh(core_axis_name="c", subcore_axis_name="s")
    run = pl.kernel(
        _kmink_body,
        out_type=jax.ShapeDtypeStruct((B, C, 2 * K), jnp.float32),
        mesh=mesh,
        scratch_types=[
            pltpu.VMEM((CH, CB), jnp.float32),
            pltpu.VMEM((CH, CB), jnp.float32),
            pltpu.VMEM((GPT, 2 * K, L), jnp.float32),
            pltpu.VMEM((L, 2 * K), jnp.float32),
            pltpu.SemaphoreType.DMA,
            pltpu.SemaphoreType.DMA,
        ],
        compiler_params=pltpu.CompilerParams(
            use_tc_tiling_on_sc=False, needs_layout_passes=False
        ),
    )
    return run(input)


# speedup vs baseline: 1.2698x; 1.2698x over previous
import jax
import jax.numpy as jnp
from jax import lax
from jax.experimental import pallas as pl
from jax.experimental.pallas import tpu as pltpu
from jax.experimental.pallas import tpu_sc as plsc

B = 4
S = 8192
C = 768
K = 8
L = 16
NW = 32
CB = 128
GPT = CB // L
CH = 256
NCH = S // CH

_SORT8 = ((0, 1), (2, 3), (4, 5), (6, 7),
          (0, 2), (1, 3), (4, 6), (5, 7),
          (1, 2), (5, 6),
          (0, 4), (1, 5), (2, 6), (3, 7),
          (2, 4), (3, 5),
          (1, 2), (3, 4), (5, 6))

_BITONIC = ((4, (0, 1, 2, 3)), (2, (0, 1, 4, 5)), (1, (0, 2, 4, 6)))


def _sort8(vs):
    vs = list(vs)
    for a, b in _SORT8:
        lo = jnp.minimum(vs[a], vs[b])
        hi = jnp.maximum(vs[a], vs[b])
        vs[a], vs[b] = lo, hi
    return vs


def _bitonic(c, desc):
    c = list(c)
    for d, idxs in _BITONIC:
        for i in idxs:
            lo = jnp.minimum(c[i], c[i + d])
            hi = jnp.maximum(c[i], c[i + d])
            c[i], c[i + d] = (hi, lo) if desc else (lo, hi)
    return c


def _consume_chunk(buf, g, carry):
    def batch_body(bi, st):
        base = bi * 8
        s = _sort8([buf[base + i, pl.ds(g * L, L)] for i in range(8)])
        cmax = [jnp.maximum(st[i], s[i]) for i in range(K)]
        maxs = _bitonic(cmax, desc=True)
        cmin = [jnp.minimum(st[K + i], s[7 - i]) for i in range(K)]
        mins = _bitonic(cmin, desc=False)
        return tuple(maxs) + tuple(mins)

    return lax.fori_loop(0, CH // 8, batch_body, carry)


def _kmink_body(x_hbm, out_hbm, buf0, buf1, state, obuf, sem0, sem1):
    wid = lax.axis_index("s") * 2 + lax.axis_index("c")
    tid = wid % 24
    b = tid // 6
    cb = tid % 6
    lane = jnp.arange(L, dtype=jnp.int32)
    bufs = (buf0, buf1)
    sems = (sem0, sem1)

    def src(ci):
        return x_hbm.at[b, pl.ds(ci * CH, CH), pl.ds(cb * CB, CB)]

    neg_inf = jnp.full((L,), -jnp.inf, jnp.float32)
    pos_inf = jnp.full((L,), jnp.inf, jnp.float32)

    def init_state(g, _):
        for i in range(K):
            state[g, i] = neg_inf
            state[g, K + i] = pos_inf
        return 0

    lax.fori_loop(0, GPT, init_state, 0)

    pltpu.async_copy(src(0), bufs[0], sems[0]).wait()

    def do_chunk_pair(ci, _):
        for bb in range(2):
            ce = ci * 2 + bb
            nxt = 1 - bb
            have_next = ce + 1 < NCH

            @pl.when(have_next)
            def _():
                pltpu.async_copy(src(ce + 1), bufs[nxt], sems[nxt])

            def do_group(g, _):
                carry = tuple(state[g, i] for i in range(2 * K))
                carry = _consume_chunk(bufs[bb], g, carry)
                for i in range(2 * K):
                    state[g, i] = carry[i]
                return 0

            lax.fori_loop(0, GPT, do_group, 0)

            @pl.when(have_next)
            def _():
                pltpu.make_async_copy(src(ce + 1), bufs[nxt], sems[nxt]).wait()
        return 0

    lax.fori_loop(0, NCH // 2, do_chunk_pair, 0)

    def write_group(g, _):
        for i in range(K):
            plsc.store_scatter(obuf, [lane, jnp.full((L,), i, jnp.int32)],
                               state[g, K + i])
            plsc.store_scatter(obuf, [lane, jnp.full((L,), K + i, jnp.int32)],
                               state[g, i])
        pltpu.sync_copy(obuf, out_hbm.at[b, pl.ds(cb * CB + g * L, L), :])
        return 0

    lax.fori_loop(0, GPT, write_group, 0)


@jax.jit
def kernel(input):
    mesh = plsc.VectorSubcoreMesh(core_axis_name="c", subcore_axis_name="s")
    run = pl.kernel(
        _kmink_body,
        out_type=jax.ShapeDtypeStruct((B, C, 2 * K), jnp.float32),
        mesh=mesh,
        scratch_types=[
            pltpu.VMEM((CH, CB), jnp.float32),
            pltpu.VMEM((CH, CB), jnp.float32),
            pltpu.VMEM((GPT, 2 * K, L), jnp.float32),
            pltpu.VMEM((L, 2 * K), jnp.float32),
            pltpu.SemaphoreType.DMA,
            pltpu.SemaphoreType.DMA,
        ],
        compiler_params=pltpu.CompilerParams(
            use_tc_tiling_on_sc=True, needs_layout_passes=False
        ),
    )
    return run(input)


# speedup vs baseline: 1.5294x; 1.2045x over previous
import jax
import jax.numpy as jnp
from jax import lax
from jax.experimental import pallas as pl
from jax.experimental.pallas import tpu as pltpu
from jax.experimental.pallas import tpu_sc as plsc

B = 4
S = 8192
C = 768
K = 8
L = 16
CBW = 128
NCB = C // CBW
NSQ = 4
SQ = S // NSQ
GPT = CBW // L
CH = 256
NCH = SQ // CH
TPS = 2 * NCB * NSQ
MPS = 2 * NCB * GPT

_SORT8 = ((0, 1), (2, 3), (4, 5), (6, 7),
          (0, 2), (1, 3), (4, 6), (5, 7),
          (1, 2), (5, 6),
          (0, 4), (1, 5), (2, 6), (3, 7),
          (2, 4), (3, 5),
          (1, 2), (3, 4), (5, 6))

_BITONIC = ((4, (0, 1, 2, 3)), (2, (0, 1, 4, 5)), (1, (0, 2, 4, 6)))


def _sort8(vs):
    vs = list(vs)
    for a, b in _SORT8:
        lo = jnp.minimum(vs[a], vs[b])
        hi = jnp.maximum(vs[a], vs[b])
        vs[a], vs[b] = lo, hi
    return vs


def _bitonic(c, desc):
    c = list(c)
    for d, idxs in _BITONIC:
        for i in idxs:
            lo = jnp.minimum(c[i], c[i + d])
            hi = jnp.maximum(c[i], c[i + d])
            c[i], c[i + d] = (hi, lo) if desc else (lo, hi)
    return c


def _merge_top(lst, other_rev, desc):
    op = jnp.maximum if desc else jnp.minimum
    c = [op(lst[i], other_rev[i]) for i in range(K)]
    return _bitonic(c, desc=desc)


def _consume_chunk(buf, g, carry):
    def batch_body(bi, st):
        base = bi * 8
        s = _sort8([buf[base + i, pl.ds(g * L, L)] for i in range(8)])
        maxs = _merge_top(list(st[:K]), s, desc=True)
        mins = _merge_top(list(st[K:]), s[::-1], desc=False)
        return tuple(maxs) + tuple(mins)

    return lax.fori_loop(0, CH // 8, batch_body, carry)


def _kmink_body(x_hbm, out_hbm, buf0, buf1, state, mbuf, obuf, shared,
                sem0, sem1):
    cid = lax.axis_index("c")
    sid = lax.axis_index("s")
    lane = jnp.arange(L, dtype=jnp.int32)
    bufs = (buf0, buf1)
    sems = (sem0, sem1)

    neg_inf = jnp.full((L,), -jnp.inf, jnp.float32)
    pos_inf = jnp.full((L,), jnp.inf, jnp.float32)

    def do_task(k, _):
        t = sid + 16 * k
        b_local = t // (NCB * NSQ)
        rem = t % (NCB * NSQ)
        cb = rem // NSQ
        sq = rem % NSQ
        b = 2 * cid + b_local
        row0 = sq * SQ

        def src(ci):
            return x_hbm.at[b, pl.ds(row0 + ci * CH, CH),
                            pl.ds(cb * CBW, CBW)]

        def init_state(g, _):
            for i in range(K):
                state[i, pl.ds(g * L, L)] = neg_inf
                state[K + i, pl.ds(g * L, L)] = pos_inf
            return 0

        lax.fori_loop(0, GPT, init_state, 0)

        pltpu.async_copy(src(0), bufs[0], sems[0]).wait()

        def do_chunk_pair(ci, _):
            for bb in range(2):
                ce = ci * 2 + bb
                nxt = 1 - bb
                have_next = ce + 1 < NCH

                @pl.when(have_next)
                def _():
                    pltpu.async_copy(src(ce + 1), bufs[nxt], sems[nxt])

                def do_group(g, _):
                    carry = tuple(state[i, pl.ds(g * L, L)]
                                  for i in range(2 * K))
                    carry = _consume_chunk(bufs[bb], g, carry)
                    for i in range(2 * K):
                        state[i, pl.ds(g * L, L)] = carry[i]
                    return 0

                lax.fori_loop(0, GPT, do_group, 0)

                @pl.when(have_next)
                def _():
                    pltpu.make_async_copy(src(ce + 1), bufs[nxt],
                                          sems[nxt]).wait()
            return 0

        lax.fori_loop(0, NCH // 2, do_chunk_pair, 0)
        pltpu.sync_copy(state, shared.at[t])
        return 0

    lax.fori_loop(0, TPS // 16, do_task, 0)
    plsc.subcore_barrier()

    def do_merge(k, _):
        u = sid + 16 * k
        b_local = u // (NCB * GPT)
        rem = u % (NCB * GPT)
        cb = rem // GPT
        g = rem % GPT
        b = 2 * cid + b_local

        for sq in range(NSQ):
            t = b_local * (NCB * NSQ) + cb * NSQ + sq
            pltpu.sync_copy(shared.at[t], mbuf.at[sq])

        maxs = [mbuf[0, i, pl.ds(g * L, L)] for i in range(K)]
        mins = [mbuf[0, K + i, pl.ds(g * L, L)] for i in range(K)]
        for sq in range(1, NSQ):
            pmax = [mbuf[sq, i, pl.ds(g * L, L)] for i in range(K)]
            pmin = [mbuf[sq, K + i, pl.ds(g * L, L)] for i in range(K)]
            maxs = _merge_top(maxs, pmax[::-1], desc=True)
            mins = _merge_top(mins, pmin[::-1], desc=False)

        for i in range(K):
            plsc.store_scatter(obuf, [lane, jnp.full((L,), i, jnp.int32)],
                               mins[i])
            plsc.store_scatter(obuf, [lane, jnp.full((L,), K + i, jnp.int32)],
                               maxs[i])
        pltpu.sync_copy(obuf, out_hbm.at[b, pl.ds(cb * CBW + g * L, L), :])
        return 0

    lax.fori_loop(0, MPS // 16, do_merge, 0)


@jax.jit
def kernel(input):
    mesh = plsc.VectorSubcoreMesh(core_axis_name="c", subcore_axis_name="s")
    run = pl.kernel(
        _kmink_body,
        out_type=jax.ShapeDtypeStruct((B, C, 2 * K), jnp.float32),
        mesh=mesh,
        scratch_types=[
            pltpu.VMEM((CH, CBW), jnp.float32),
            pltpu.VMEM((CH, CBW), jnp.float32),
            pltpu.VMEM((2 * K, CBW), jnp.float32),
            pltpu.VMEM((NSQ, 2 * K, CBW), jnp.float32),
            pltpu.VMEM((L, 2 * K), jnp.float32),
            pltpu.VMEM_SHARED((TPS, 2 * K, CBW), jnp.float32),
            pltpu.SemaphoreType.DMA,
            pltpu.SemaphoreType.DMA,
        ],
        compiler_params=pltpu.CompilerParams(
            use_tc_tiling_on_sc=True, needs_layout_passes=False
        ),
    )
    return run(input)
